# baseline (speedup 1.0000x reference)
import functools

import jax
import jax.numpy as jnp
from jax import lax
from jax.experimental import pallas as pl
from jax.experimental.pallas import tpu as pltpu
from jax.experimental.pallas import tpu_sc as plsc

_LANES = 16
_NWORKERS = 32
_CHUNK = 16
_LN_EPS = 1e-12


_GATHER_DNUMS = lax.GatherDimensionNumbers(
    offset_dims=(), collapsed_slice_dims=(0,), start_index_map=(0,))


def _lane_shuffle(x, idx):
    return lax.gather(x, idx[:, None], _GATHER_DNUMS, slice_sizes=(1,),
                      mode=lax.GatherScatterMode.PROMISE_IN_BOUNDS)


def _allreduce16(x):
    iota = lax.iota(jnp.int32, _LANES)
    for sh in (8, 4, 2, 1):
        x = x + _lane_shuffle(x, iota ^ sh)
    return x


def _rsqrt16(x):
    i = plsc.bitcast(x, jnp.int32)
    y = plsc.bitcast(jnp.int32(0x5F3759DF) - (i >> 1), jnp.float32)
    half_x = x * jnp.float32(0.5)
    y = y * (jnp.float32(1.5) - half_x * y * y)
    y = y * (jnp.float32(1.5) - half_x * y * y)
    y = y * (jnp.float32(1.5) - half_x * y * y)
    return y


@functools.lru_cache(maxsize=None)
def _build(n_tok, seq_len, hidden):
    spw = n_tok // _NWORKERS
    n_chunks = spw // _CHUNK
    n_vregs = hidden // _LANES
    mesh = plsc.VectorSubcoreMesh(core_axis_name="c", subcore_axis_name="s")

    @functools.partial(
        pl.kernel,
        out_type=jax.ShapeDtypeStruct((n_tok, hidden), jnp.float32),
        mesh=mesh,
        compiler_params=pltpu.CompilerParams(needs_layout_passes=False),
        scratch_types=[
            pltpu.VMEM((spw,), jnp.int32),
            pltpu.VMEM((spw,), jnp.int32),
            pltpu.VMEM((spw,), jnp.int32),
            pltpu.VMEM((_CHUNK, hidden), jnp.float32),
            pltpu.VMEM((_CHUNK, hidden), jnp.float32),
            pltpu.VMEM((_CHUNK, hidden), jnp.float32),
            pltpu.VMEM((_CHUNK, hidden), jnp.float32),
            pltpu.VMEM((hidden,), jnp.float32),
            pltpu.VMEM((hidden,), jnp.float32),
            pltpu.SemaphoreType.DMA,
        ],
    )
    def tie_kernel(ids_hbm, tt_hbm, task_hbm, wemb, pemb, temb, kemb,
                   gamma_hbm, beta_hbm, out_hbm,
                   ids_v, tt_v, task_v, wbuf, tbuf, kbuf, pbuf,
                   gamma_v, beta_v, sem):
        wid = lax.axis_index("s") * mesh.num_cores + lax.axis_index("c")
        base = wid * spw
        s_base = lax.rem(base, seq_len)

        pltpu.sync_copy(ids_hbm.at[pl.ds(base, spw)], ids_v)
        pltpu.sync_copy(tt_hbm.at[pl.ds(base, spw)], tt_v)
        pltpu.sync_copy(task_hbm.at[pl.ds(base, spw)], task_v)
        pltpu.sync_copy(gamma_hbm, gamma_v)
        pltpu.sync_copy(beta_hbm, beta_v)

        def chunk_body(c, carry):
            off = c * _CHUNK
            widx = ids_v[pl.ds(off, _CHUNK)]
            tidx = tt_v[pl.ds(off, _CHUNK)]
            kidx = task_v[pl.ds(off, _CHUNK)]
            d1 = pltpu.async_copy(wemb.at[widx], wbuf, sem)
            d2 = pltpu.async_copy(temb.at[tidx], tbuf, sem)
            d3 = pltpu.async_copy(kemb.at[kidx], kbuf, sem)
            pltpu.sync_copy(pemb.at[pl.ds(s_base + off, _CHUNK)], pbuf)
            d1.wait()
            d2.wait()
            d3.wait()

            def token_body(t, tc):
                s = jnp.zeros((_LANES,), jnp.float32)
                ss = jnp.zeros((_LANES,), jnp.float32)
                for j in range(n_vregs):
                    sl = pl.ds(j * _LANES, _LANES)
                    v = wbuf[t, sl] + tbuf[t, sl] + kbuf[t, sl] + pbuf[t, sl]
                    wbuf[t, sl] = v
                    s = s + v
                    ss = ss + v * v
                inv_h = jnp.float32(1.0 / hidden)
                mean_v = _allreduce16(s) * inv_h
                var_v = _allreduce16(ss) * inv_h - mean_v * mean_v
                rstd_v = _rsqrt16(var_v + jnp.float32(_LN_EPS))
                for j in range(n_vregs):
                    sl = pl.ds(j * _LANES, _LANES)
                    a = gamma_v[sl] * rstd_v
                    wbuf[t, sl] = (wbuf[t, sl] - mean_v) * a + beta_v[sl]
                return tc

            lax.fori_loop(0, _CHUNK, token_body, 0)
            pltpu.sync_copy(wbuf, out_hbm.at[pl.ds(base + off, _CHUNK)])
            return carry

        lax.fori_loop(0, n_chunks, chunk_body, 0)

    return tie_kernel


def kernel(input_ids, token_type_ids, task_type_ids, word_emb, pos_emb,
           type_emb, task_emb, ln_gamma, ln_beta):
    b, s = input_ids.shape
    hidden = word_emb.shape[1]
    n_tok = b * s
    fn = _build(n_tok, s, hidden)
    out = fn(input_ids.reshape(-1).astype(jnp.int32),
             token_type_ids.reshape(-1).astype(jnp.int32),
             task_type_ids.reshape(-1).astype(jnp.int32),
             word_emb, pos_emb, type_emb, task_emb, ln_gamma, ln_beta)
    return out.reshape(b, s, hidden)


# speedup vs baseline: 1.0081x; 1.0081x over previous
import functools

import jax
import jax.numpy as jnp
from jax import lax
from jax.experimental import pallas as pl
from jax.experimental.pallas import tpu as pltpu
from jax.experimental.pallas import tpu_sc as plsc

_LANES = 16
_NWORKERS = 32
_CHUNK = 16
_QUAD = 4
_LN_EPS = 1e-12

_GATHER_DNUMS = lax.GatherDimensionNumbers(
    offset_dims=(), collapsed_slice_dims=(0,), start_index_map=(0,))


def _lane_shuffle(x, idx):
    return lax.gather(x, idx[:, None], _GATHER_DNUMS, slice_sizes=(1,),
                      mode=lax.GatherScatterMode.PROMISE_IN_BOUNDS)


def _allreduce16(x):
    iota = lax.iota(jnp.int32, _LANES)
    for sh in (8, 4, 2, 1):
        x = x + _lane_shuffle(x, iota ^ sh)
    return x


def _rsqrt16(x):
    i = plsc.bitcast(x, jnp.int32)
    y = plsc.bitcast(jnp.int32(0x5F3759DF) - (i >> 1), jnp.float32)
    half_x = x * jnp.float32(0.5)
    y = y * (jnp.float32(1.5) - half_x * y * y)
    y = y * (jnp.float32(1.5) - half_x * y * y)
    y = y * (jnp.float32(1.5) - half_x * y * y)
    return y


@functools.lru_cache(maxsize=None)
def _build(n_tok, seq_len, hidden):
    spw = n_tok // _NWORKERS
    n_pairs = spw // (2 * _CHUNK)
    nv = hidden // _LANES
    mesh = plsc.VectorSubcoreMesh(core_axis_name="c", subcore_axis_name="s")
    buf_t = pltpu.VMEM((_CHUNK, hidden), jnp.float32)

    @functools.partial(
        pl.kernel,
        out_type=jax.ShapeDtypeStruct((n_tok, hidden), jnp.float32),
        mesh=mesh,
        compiler_params=pltpu.CompilerParams(needs_layout_passes=False),
        scratch_types=[
            pltpu.VMEM((spw,), jnp.int32),
            pltpu.VMEM((spw,), jnp.int32),
            pltpu.VMEM((spw,), jnp.int32),
            buf_t, buf_t, buf_t, buf_t,
            buf_t, buf_t, buf_t, buf_t,
            pltpu.VMEM((hidden,), jnp.float32),
            pltpu.VMEM((hidden,), jnp.float32),
            pltpu.SemaphoreType.DMA,
        ],
    )
    def tie_kernel(ids_hbm, tt_hbm, task_hbm, wemb, pemb, temb, kemb,
                   gamma_hbm, beta_hbm, out_hbm,
                   ids_v, tt_v, task_v,
                   wa, ta, ka, pa, wb, tb, kb, pb,
                   gamma_v, beta_v, sem):
        wid = lax.axis_index("s") * mesh.num_cores + lax.axis_index("c")
        base = wid * spw
        s_base = lax.rem(base, seq_len)

        pltpu.sync_copy(ids_hbm.at[pl.ds(base, spw)], ids_v)
        pltpu.sync_copy(tt_hbm.at[pl.ds(base, spw)], tt_v)
        pltpu.sync_copy(task_hbm.at[pl.ds(base, spw)], task_v)
        pltpu.sync_copy(gamma_hbm, gamma_v)
        pltpu.sync_copy(beta_hbm, beta_v)

        def issue(c, w, t, k, p):
            off = pl.multiple_of(c * _CHUNK, _CHUNK)
            pltpu.async_copy(wemb.at[ids_v[pl.ds(off, _CHUNK)]], w, sem)
            pltpu.async_copy(temb.at[tt_v[pl.ds(off, _CHUNK)]], t, sem)
            pltpu.async_copy(kemb.at[task_v[pl.ds(off, _CHUNK)]], k, sem)
            pltpu.async_copy(pemb.at[pl.ds(s_base + off, _CHUNK)], p, sem)

        def wait4(w):
            for _ in range(4):
                pltpu.make_async_copy(pemb.at[pl.ds(0, _CHUNK)], w, sem).wait()

        zz = jnp.zeros((_LANES,), jnp.float32)
        inv_h = jnp.float32(1.0 / hidden)

        def compute(c, w, t, k, p):
            off = pl.multiple_of(c * _CHUNK, _CHUNK)

            def quad_body(q, carry):
                t0 = q * _QUAD
                stats = []
                for dt in range(_QUAD):
                    tk = t0 + dt
                    s = zz
                    ss = zz
                    for j in range(nv):
                        sl = pl.ds(j * _LANES, _LANES)
                        v = w[tk, sl] + t[tk, sl] + k[tk, sl] + p[tk, sl]
                        w[tk, sl] = v
                        s = s + v
                        ss = ss + v * v
                    mean_v = _allreduce16(s) * inv_h
                    var_v = _allreduce16(ss) * inv_h - mean_v * mean_v
                    rstd_v = _rsqrt16(var_v + jnp.float32(_LN_EPS))
                    stats.append((mean_v, rstd_v))
                for j in range(nv):
                    sl = pl.ds(j * _LANES, _LANES)
                    g = gamma_v[sl]
                    b = beta_v[sl]
                    for dt in range(_QUAD):
                        tk = t0 + dt
                        mean_v, rstd_v = stats[dt]
                        a = g * rstd_v
                        w[tk, sl] = (w[tk, sl] - mean_v) * a + b
                return carry

            lax.fori_loop(0, _CHUNK // _QUAD, quad_body, 0)
            pltpu.sync_copy(w, out_hbm.at[pl.ds(base + off, _CHUNK)])

        issue(0, wa, ta, ka, pa)

        def pair_body(cp, carry):
            c0 = cp * 2
            wait4(wa)
            issue(c0 + 1, wb, tb, kb, pb)
            compute(c0, wa, ta, ka, pa)
            wait4(wb)

            @pl.when(cp + 1 < n_pairs)
            def _():
                issue(c0 + 2, wa, ta, ka, pa)

            compute(c0 + 1, wb, tb, kb, pb)
            return carry

        lax.fori_loop(0, n_pairs, pair_body, 0)

    return tie_kernel


def kernel(input_ids, token_type_ids, task_type_ids, word_emb, pos_emb,
           type_emb, task_emb, ln_gamma, ln_beta):
    b, s = input_ids.shape
    hidden = word_emb.shape[1]
    n_tok = b * s
    fn = _build(n_tok, s, hidden)
    out = fn(input_ids.reshape(-1).astype(jnp.int32),
             token_type_ids.reshape(-1).astype(jnp.int32),
             task_type_ids.reshape(-1).astype(jnp.int32),
             word_emb, pos_emb, type_emb, task_emb, ln_gamma, ln_beta)
    return out.reshape(b, s, hidden)
